# baseline (speedup 1.0000x reference)
import functools

import jax
import jax.numpy as jnp
from jax import lax
from jax.experimental import pallas as pl
from jax.experimental.pallas import tpu as pltpu
from jax.experimental.pallas import tpu_sc as plsc

TOKENS = 16384
DIM = 2048
N_EXPERTS = 64
TOPK = 8
BLOCK_T = 2048

NUM_CORES = 2
NUM_SUBCORES = 16
LANES = 16
NUM_WORKERS = NUM_CORES * NUM_SUBCORES
ROWS_PER_W = TOKENS // NUM_WORKERS
GROUPS = ROWS_PER_W // LANES
INT_MIN = -(2**31)


def _dense_kernel(x_ref, w_ref, probs_out_ref, probs_t_out_ref):
    x = x_ref[...]
    w = w_ref[...]
    logits = jax.lax.dot_general(
        x, w,
        dimension_numbers=(((1,), (1,)), ((), ())),
        preferred_element_type=jnp.float32,
    )
    m = jnp.max(logits, axis=-1, keepdims=True)
    e = jnp.exp(logits - m)
    probs = e * (1.0 / jnp.sum(e, axis=-1, keepdims=True))
    probs_out_ref[...] = probs
    probs_t_out_ref[...] = probs.T


def _tree_max(vals):
    vals = list(vals)
    while len(vals) > 1:
        nxt = [jnp.maximum(vals[i], vals[i + 1]) for i in range(0, len(vals) - 1, 2)]
        if len(vals) % 2:
            nxt.append(vals[-1])
        vals = nxt
    return vals[0]


def _route_kernel(probs_t_hbm, w_t_hbm, i_t_hbm, pbuf, wbuf, ibuf):
    wid = lax.axis_index("s") * NUM_CORES + lax.axis_index("c")
    base = wid * ROWS_PER_W
    pltpu.sync_copy(probs_t_hbm.at[:, pl.ds(base, ROWS_PER_W)], pbuf)

    def body(j, carry):
        sl = pl.ds(j * LANES, LANES)
        keys = []
        for e in range(N_EXPERTS):
            ib = lax.bitcast_convert_type(pbuf[e, sl], jnp.int32)
            keys.append((ib | 63) - e)
        slots, nexts = [], []
        for p in range(0, N_EXPERTS, 2):
            a, b = keys[p], keys[p + 1]
            slots.append(jnp.maximum(a, b))
            nexts.append(jnp.minimum(a, b))
        for k in range(TOPK):
            mx = _tree_max(slots)
            ibuf[k, sl] = 63 - (mx & 63)
            wbuf[k, sl] = lax.bitcast_convert_type(mx & ~63, jnp.float32)
            if k < TOPK - 1:
                hit = [s == mx for s in slots]
                slots = [jnp.where(h, n, s)
                         for h, s, n in zip(hit, slots, nexts)]
                nexts = [jnp.where(h, INT_MIN, n)
                         for h, n in zip(hit, nexts)]
        return carry

    lax.fori_loop(0, GROUPS, body, 0, unroll=4)
    pltpu.sync_copy(wbuf, w_t_hbm.at[:, pl.ds(base, ROWS_PER_W)])
    pltpu.sync_copy(ibuf, i_t_hbm.at[:, pl.ds(base, ROWS_PER_W)])


_route = functools.partial(
    pl.kernel,
    mesh=plsc.VectorSubcoreMesh(core_axis_name="c", subcore_axis_name="s"),
    out_type=(
        jax.ShapeDtypeStruct((TOPK, TOKENS), jnp.float32),
        jax.ShapeDtypeStruct((TOPK, TOKENS), jnp.int32),
    ),
    scratch_types=[
        pltpu.VMEM((N_EXPERTS, ROWS_PER_W), jnp.float32),
        pltpu.VMEM((TOPK, ROWS_PER_W), jnp.float32),
        pltpu.VMEM((TOPK, ROWS_PER_W), jnp.int32),
    ],
)(_route_kernel)


@jax.jit
def kernel(x, W):
    probs, probs_t = pl.pallas_call(
        _dense_kernel,
        grid=(TOKENS // BLOCK_T,),
        in_specs=[
            pl.BlockSpec((BLOCK_T, DIM), lambda i: (i, 0)),
            pl.BlockSpec((N_EXPERTS, DIM), lambda i: (0, 0)),
        ],
        out_specs=(
            pl.BlockSpec((BLOCK_T, N_EXPERTS), lambda i: (i, 0)),
            pl.BlockSpec((N_EXPERTS, BLOCK_T), lambda i: (0, i)),
        ),
        out_shape=(
            jax.ShapeDtypeStruct((TOKENS, N_EXPERTS), jnp.float32),
            jax.ShapeDtypeStruct((N_EXPERTS, TOKENS), jnp.float32),
        ),
        compiler_params=pltpu.CompilerParams(
            dimension_semantics=("parallel",),
        ),
    )(x, W)
    weights_t, indices_t = _route(probs_t)
    return (weights_t.T, indices_t.T, probs)


# speedup vs baseline: 1.2083x; 1.2083x over previous
import functools

import jax
import jax.numpy as jnp
from jax import lax
from jax.experimental import pallas as pl
from jax.experimental.pallas import tpu as pltpu
from jax.experimental.pallas import tpu_sc as plsc

TOKENS = 16384
DIM = 2048
N_EXPERTS = 64
TOPK = 8
BLOCK_T = 2048

NUM_CORES = 2
NUM_SUBCORES = 16
LANES = 16
NUM_WORKERS = NUM_CORES * NUM_SUBCORES
ROWS_PER_W = TOKENS // NUM_WORKERS
GROUPS = ROWS_PER_W // LANES
INT_MIN = -(2**31)


def _dense_kernel(x_ref, w_ref, probs_out_ref, probs_t_out_ref):
    x = x_ref[...]
    w = w_ref[...]
    logits = jax.lax.dot_general(
        x, w,
        dimension_numbers=(((1,), (1,)), ((), ())),
        preferred_element_type=jnp.float32,
    )
    m = jnp.max(logits, axis=-1, keepdims=True)
    e = jnp.exp(logits - m)
    probs = e * (1.0 / jnp.sum(e, axis=-1, keepdims=True))
    probs_out_ref[...] = probs
    probs_t_out_ref[...] = probs.T


def _tree_max(vals):
    vals = list(vals)
    while len(vals) > 1:
        nxt = [jnp.maximum(vals[i], vals[i + 1]) for i in range(0, len(vals) - 1, 2)]
        if len(vals) % 2:
            nxt.append(vals[-1])
        vals = nxt
    return vals[0]


def _route_kernel(probs_t_hbm, w_t_hbm, i_t_hbm, pbuf, wbuf, ibuf):
    wid = lax.axis_index("s") * NUM_CORES + lax.axis_index("c")
    base = wid * ROWS_PER_W
    pltpu.sync_copy(probs_t_hbm.at[:, pl.ds(base, ROWS_PER_W)], pbuf)

    def body(j, carry):
        sl = pl.ds(j * LANES, LANES)
        keys = []
        for e in range(N_EXPERTS):
            ib = lax.bitcast_convert_type(pbuf[e, sl], jnp.int32)
            keys.append((ib | 63) - e)
        slots, nexts = [], []
        for p in range(0, N_EXPERTS, 2):
            a, b = keys[p], keys[p + 1]
            slots.append(jnp.maximum(a, b))
            nexts.append(jnp.minimum(a, b))
        for k in range(TOPK):
            mx = _tree_max(slots)
            ibuf[k, sl] = 63 - (mx & 63)
            wbuf[k, sl] = lax.bitcast_convert_type(mx & ~63, jnp.float32)
            if k < TOPK - 1:
                hit = [s == mx for s in slots]
                slots = [jnp.where(h, n, s)
                         for h, s, n in zip(hit, slots, nexts)]
                nexts = [jnp.where(h, INT_MIN, n)
                         for h, n in zip(hit, nexts)]
        return carry

    lax.fori_loop(0, GROUPS, body, 0, unroll=1)
    pltpu.sync_copy(wbuf, w_t_hbm.at[:, pl.ds(base, ROWS_PER_W)])
    pltpu.sync_copy(ibuf, i_t_hbm.at[:, pl.ds(base, ROWS_PER_W)])


_route = functools.partial(
    pl.kernel,
    mesh=plsc.VectorSubcoreMesh(core_axis_name="c", subcore_axis_name="s"),
    out_type=(
        jax.ShapeDtypeStruct((TOPK, TOKENS), jnp.float32),
        jax.ShapeDtypeStruct((TOPK, TOKENS), jnp.int32),
    ),
    scratch_types=[
        pltpu.VMEM((N_EXPERTS, ROWS_PER_W), jnp.float32),
        pltpu.VMEM((TOPK, ROWS_PER_W), jnp.float32),
        pltpu.VMEM((TOPK, ROWS_PER_W), jnp.int32),
    ],
)(_route_kernel)


@jax.jit
def kernel(x, W):
    probs, probs_t = pl.pallas_call(
        _dense_kernel,
        grid=(TOKENS // BLOCK_T,),
        in_specs=[
            pl.BlockSpec((BLOCK_T, DIM), lambda i: (i, 0)),
            pl.BlockSpec((N_EXPERTS, DIM), lambda i: (0, 0)),
        ],
        out_specs=(
            pl.BlockSpec((BLOCK_T, N_EXPERTS), lambda i: (i, 0)),
            pl.BlockSpec((N_EXPERTS, BLOCK_T), lambda i: (0, i)),
        ),
        out_shape=(
            jax.ShapeDtypeStruct((TOKENS, N_EXPERTS), jnp.float32),
            jax.ShapeDtypeStruct((N_EXPERTS, TOKENS), jnp.float32),
        ),
        compiler_params=pltpu.CompilerParams(
            dimension_semantics=("parallel",),
        ),
    )(x, W)
    weights_t, indices_t = _route(probs_t)
    return (weights_t.T, indices_t.T, probs)


# speedup vs baseline: 1.2132x; 1.0040x over previous
import functools

import jax
import jax.numpy as jnp
from jax import lax
from jax.experimental import pallas as pl
from jax.experimental.pallas import tpu as pltpu
from jax.experimental.pallas import tpu_sc as plsc

TOKENS = 16384
DIM = 2048
N_EXPERTS = 64
TOPK = 8
BLOCK_T = 2048

NUM_CORES = 2
NUM_SUBCORES = 16
LANES = 16
NUM_WORKERS = NUM_CORES * NUM_SUBCORES
ROWS_PER_W = TOKENS // NUM_WORKERS
GROUPS = ROWS_PER_W // LANES
INT_MIN = -(2**31)


def _dense_kernel(x_ref, w_ref, probs_out_ref, probs_t_out_ref):
    x = x_ref[...]
    w = w_ref[...]
    logits = jax.lax.dot_general(
        x, w,
        dimension_numbers=(((1,), (1,)), ((), ())),
        preferred_element_type=jnp.float32,
    )
    m = jnp.max(logits, axis=-1, keepdims=True)
    e = jnp.exp(logits - m)
    probs = e * (1.0 / jnp.sum(e, axis=-1, keepdims=True))
    probs_out_ref[...] = probs
    probs_t_out_ref[...] = probs.T


def _tree_max(vals):
    vals = list(vals)
    while len(vals) > 1:
        nxt = [jnp.maximum(vals[i], vals[i + 1]) for i in range(0, len(vals) - 1, 2)]
        if len(vals) % 2:
            nxt.append(vals[-1])
        vals = nxt
    return vals[0]


def _route_kernel(probs_t_hbm, w_t_hbm, i_t_hbm, pbuf, wbuf, ibuf):
    wid = lax.axis_index("s") * NUM_CORES + lax.axis_index("c")
    base = wid * ROWS_PER_W
    pltpu.sync_copy(probs_t_hbm.at[:, pl.ds(base, ROWS_PER_W)], pbuf)

    def body(j, carry):
        sl = pl.ds(j * LANES, LANES)
        keys = []
        for e in range(N_EXPERTS):
            ib = lax.bitcast_convert_type(pbuf[e, sl], jnp.int32)
            keys.append((ib | 63) - e)
        slots, nexts = [], []
        for p in range(0, N_EXPERTS, 2):
            a, b = keys[p], keys[p + 1]
            slots.append(jnp.maximum(a, b))
            nexts.append(jnp.minimum(a, b))
        for k in range(TOPK):
            mx = _tree_max(slots)
            ibuf[k, sl] = 63 - (mx & 63)
            wbuf[k, sl] = lax.bitcast_convert_type(mx & ~63, jnp.float32)
            if k < TOPK - 1:
                hit = [s == mx for s in slots]
                slots = [jnp.where(h, n, s)
                         for h, s, n in zip(hit, slots, nexts)]
                nexts = [jnp.where(h, INT_MIN, n)
                         for h, n in zip(hit, nexts)]
        return carry

    lax.fori_loop(0, GROUPS, body, 0, unroll=2)
    pltpu.sync_copy(wbuf, w_t_hbm.at[:, pl.ds(base, ROWS_PER_W)])
    pltpu.sync_copy(ibuf, i_t_hbm.at[:, pl.ds(base, ROWS_PER_W)])


_route = functools.partial(
    pl.kernel,
    mesh=plsc.VectorSubcoreMesh(core_axis_name="c", subcore_axis_name="s"),
    out_type=(
        jax.ShapeDtypeStruct((TOPK, TOKENS), jnp.float32),
        jax.ShapeDtypeStruct((TOPK, TOKENS), jnp.int32),
    ),
    scratch_types=[
        pltpu.VMEM((N_EXPERTS, ROWS_PER_W), jnp.float32),
        pltpu.VMEM((TOPK, ROWS_PER_W), jnp.float32),
        pltpu.VMEM((TOPK, ROWS_PER_W), jnp.int32),
    ],
)(_route_kernel)


@jax.jit
def kernel(x, W):
    probs, probs_t = pl.pallas_call(
        _dense_kernel,
        grid=(TOKENS // BLOCK_T,),
        in_specs=[
            pl.BlockSpec((BLOCK_T, DIM), lambda i: (i, 0)),
            pl.BlockSpec((N_EXPERTS, DIM), lambda i: (0, 0)),
        ],
        out_specs=(
            pl.BlockSpec((BLOCK_T, N_EXPERTS), lambda i: (i, 0)),
            pl.BlockSpec((N_EXPERTS, BLOCK_T), lambda i: (0, i)),
        ),
        out_shape=(
            jax.ShapeDtypeStruct((TOKENS, N_EXPERTS), jnp.float32),
            jax.ShapeDtypeStruct((N_EXPERTS, TOKENS), jnp.float32),
        ),
        compiler_params=pltpu.CompilerParams(
            dimension_semantics=("parallel",),
        ),
    )(x, W)
    weights_t, indices_t = _route(probs_t)
    return (weights_t.T, indices_t.T, probs)
